# baseline (speedup 1.0000x reference)
import functools

import jax
import jax.numpy as jnp
from jax import lax
from jax.experimental import pallas as pl
from jax.experimental.pallas import tpu as pltpu
from jax.experimental.pallas import tpu_sc as plsc

_EPS = 1e-08
_N_ROWS = 16384
_N_FEAT = 100
_N_CAT = 6
_BT = 2048
_TPB = _BT // 128
_NT = _N_ROWS // 128
_NW = 32
_RPW = _N_ROWS // _NW
_TPW = _RPW // 128

_LN2 = 0.6931471805599453


def _mm_body(w_ref, x_ref, b_ref, g_ref, x1_ref, z_ref):
    x_r = x_ref[...].reshape(_TPB, 128, _N_FEAT)
    a1 = lax.dot_general(x_r, w_ref[...], (((2,), (1,)), ((), ())))
    at = jnp.swapaxes(a1, 1, 2) + b_ref[...]
    pad = jnp.zeros((_TPB, 8 - _N_CAT, 128), jnp.float32)
    x1 = jnp.concatenate([at, pad], axis=1)
    x1_ref[...] = x1
    z_ref[...] = x1 + g_ref[...]


def _log_poly(s):
    bits = lax.bitcast_convert_type(s, jnp.int32)
    e = ((bits >> 23) - 127).astype(jnp.float32)
    m = lax.bitcast_convert_type((bits & 0x7FFFFF) | 0x3F800000, jnp.float32)
    u = (m - 1.0) / (m + 1.0)
    u2 = u * u
    poly = 2.0 * u * (1.0 + u2 * (1.0 / 3.0 + u2 * (1.0 / 5.0
                      + u2 * (1.0 / 7.0 + u2 * (1.0 / 9.0)))))
    return poly + e * _LN2


def _sc_body(x1_hbm, z_hbm, elig_hbm, a_hbm, ent_hbm,
             x1_v, z_v, elig_v, a_v, ent_v):
    wid = lax.axis_index("s") * 2 + lax.axis_index("c")
    base = wid * _RPW
    pltpu.sync_copy(x1_hbm.at[pl.ds(wid * _TPW, _TPW)], x1_v)
    pltpu.sync_copy(z_hbm.at[pl.ds(wid * _TPW, _TPW)], z_v)

    lane = lax.iota(jnp.int32, 16)

    def tile128(t, ent_carry):
        def group(sub, ent_acc):
            off = sub * 16
            l = [x1_v[t, k, pl.ds(off, 16)] for k in range(_N_CAT)]
            z = [z_v[t, k, pl.ds(off, 16)] for k in range(_N_CAT)]

            m = l[0]
            for k in range(1, _N_CAT):
                m = jnp.maximum(m, l[k])
            d = [l[k] - m for k in range(_N_CAT)]
            e = [jnp.exp(d[k]) for k in range(_N_CAT)]
            s = e[0]
            for k in range(1, _N_CAT):
                s = s + e[k]
            ls = _log_poly(s)
            r = m + ls

            best = z[0]
            idx = jnp.zeros((16,), jnp.int32)
            for k in range(1, _N_CAT):
                take = z[k] > best
                best = jnp.maximum(best, z[k])
                idx = jnp.where(take, jnp.full((16,), k, jnp.int32), idx)

            rows = t * 128 + off + lane
            lsel = jnp.zeros((16,), jnp.float32)
            ed = jnp.zeros((16,), jnp.float32)
            dsum = jnp.zeros((16,), jnp.float32)
            for k in range(_N_CAT):
                a_k = jnp.where(idx == k, 1.0, 0.0)
                plsc.store_scatter(
                    a_v, [rows, jnp.full((16,), k, jnp.int32)], a_k)
                lsel = lsel + a_k * l[k]
                ed = ed + e[k] * d[k]
                dsum = dsum + d[k]
            elig_v[pl.ds(t * 128 + off, 16)] = lsel - r
            ent = ent_acc + (ed / s - ls) + _EPS * (dsum - 6.0 * ls)
            return ent

        return lax.fori_loop(0, 8, group, ent_carry)

    ent_acc = lax.fori_loop(0, _TPW, tile128,
                            jnp.zeros((16,), jnp.float32))
    ent_v[...] = -ent_acc

    pltpu.sync_copy(elig_v, elig_hbm.at[pl.ds(base, _RPW)])
    pltpu.sync_copy(a_v, a_hbm.at[pl.ds(base, _RPW), :])
    pltpu.sync_copy(ent_v, ent_hbm.at[pl.ds(wid * 16, 16)])


def _ent_body(p_ref, o_ref):
    o_ref[...] = jnp.sum(p_ref[...]).reshape(1, 1)


def kernel(x, W, b):
    with jax.ensure_compile_time_eval():
        g = jax.random.gumbel(jax.random.key(42), (_N_ROWS, _N_CAT),
                              jnp.float32)
        g3 = jnp.pad(g, ((0, 0), (0, 8 - _N_CAT)))
        g3 = g3.reshape(_NT, 128, 8).swapaxes(1, 2).copy()

    x1_3d, z_3d = pl.pallas_call(
        _mm_body,
        grid=(_N_ROWS // _BT,),
        in_specs=[
            pl.BlockSpec((_N_CAT, _N_FEAT), lambda i: (0, 0)),
            pl.BlockSpec((_BT, _N_FEAT), lambda i: (i, 0)),
            pl.BlockSpec((1, _N_CAT, 1), lambda i: (0, 0, 0)),
            pl.BlockSpec((_TPB, 8, 128), lambda i: (i, 0, 0)),
        ],
        out_specs=[
            pl.BlockSpec((_TPB, 8, 128), lambda i: (i, 0, 0)),
            pl.BlockSpec((_TPB, 8, 128), lambda i: (i, 0, 0)),
        ],
        out_shape=[
            jax.ShapeDtypeStruct((_NT, 8, 128), jnp.float32),
            jax.ShapeDtypeStruct((_NT, 8, 128), jnp.float32),
        ],
    )(W, x, b[None, :, None], g3)

    mesh = plsc.VectorSubcoreMesh(core_axis_name="c", subcore_axis_name="s")
    elig, a, ent_parts = pl.kernel(
        _sc_body,
        out_type=[
            jax.ShapeDtypeStruct((_N_ROWS,), jnp.float32),
            jax.ShapeDtypeStruct((_N_ROWS, _N_CAT), jnp.float32),
            jax.ShapeDtypeStruct((_NW * 16,), jnp.float32),
        ],
        mesh=mesh,
        compiler_params=pltpu.CompilerParams(needs_layout_passes=False),
        scratch_types=[
            pltpu.VMEM((_TPW, 8, 128), jnp.float32),
            pltpu.VMEM((_TPW, 8, 128), jnp.float32),
            pltpu.VMEM((_RPW,), jnp.float32),
            pltpu.VMEM((_RPW, _N_CAT), jnp.float32),
            pltpu.VMEM((16,), jnp.float32),
        ],
    )(x1_3d, z_3d)

    ent = pl.pallas_call(
        _ent_body,
        in_specs=[pl.BlockSpec((_NW * 16,), lambda: (0,))],
        out_specs=pl.BlockSpec((1, 1), lambda: (0, 0)),
        out_shape=jax.ShapeDtypeStruct((1, 1), jnp.float32),
    )(ent_parts)

    return elig, a, ent[0, 0]


# speedup vs baseline: 1.6941x; 1.6941x over previous
import functools

import jax
import jax.numpy as jnp
from jax import lax
from jax.experimental import pallas as pl
from jax.experimental.pallas import tpu as pltpu
from jax.experimental.pallas import tpu_sc as plsc

_EPS = 1e-08
_N_ROWS = 16384
_N_FEAT = 100
_N_CAT = 6
_BT = 2048
_NW = 32
_RPW = _N_ROWS // _NW
_NG = _RPW // 16

_LN2 = 0.6931471805599453


def _mm_body(w_ref, xt_ref, b_ref, d_ref):
    d_ref[...] = lax.dot_general(w_ref[...], xt_ref[...],
                                 (((1,), (0,)), ((), ()))) + b_ref[...][:, None]


def _log_poly(s):
    bits = lax.bitcast_convert_type(s, jnp.int32)
    e = ((bits >> 23) - 127).astype(jnp.float32)
    m = lax.bitcast_convert_type((bits & 0x7FFFFF) | 0x3F800000, jnp.float32)
    u = (m - 1.0) / (m + 1.0)
    u2 = u * u
    poly = 2.0 * u * (1.0 + u2 * (1.0 / 3.0 + u2 * (1.0 / 5.0
                      + u2 * (1.0 / 7.0 + u2 * (1.0 / 9.0)))))
    return poly + e * _LN2


def _sc_body(d_hbm, gt_hbm, elig_hbm, at_hbm, ent_hbm,
             d_v, g_v, elig_v, at_v, ent_v):
    wid = lax.axis_index("s") * 2 + lax.axis_index("c")
    base = wid * _RPW
    pltpu.sync_copy(d_hbm.at[:, pl.ds(base, _RPW)], d_v)
    pltpu.sync_copy(gt_hbm.at[:, pl.ds(base, _RPW)], g_v)

    def group(i, ent_acc):
        off = i * 16
        l = [d_v[k, pl.ds(off, 16)] for k in range(_N_CAT)]
        z = [l[k] + g_v[k, pl.ds(off, 16)] for k in range(_N_CAT)]

        m = l[0]
        for k in range(1, _N_CAT):
            m = jnp.maximum(m, l[k])
        d = [l[k] - m for k in range(_N_CAT)]
        e = [jnp.exp(d[k]) for k in range(_N_CAT)]
        s = e[0]
        for k in range(1, _N_CAT):
            s = s + e[k]
        ls = _log_poly(s)
        r = m + ls

        best = z[0]
        idx = jnp.zeros((16,), jnp.int32)
        for k in range(1, _N_CAT):
            take = z[k] > best
            best = jnp.maximum(best, z[k])
            idx = jnp.where(take, jnp.full((16,), k, jnp.int32), idx)

        lsel = jnp.zeros((16,), jnp.float32)
        ed = jnp.zeros((16,), jnp.float32)
        dsum = jnp.zeros((16,), jnp.float32)
        for k in range(_N_CAT):
            a_k = jnp.where(idx == k, 1.0, 0.0)
            at_v[k, pl.ds(off, 16)] = a_k
            lsel = lsel + a_k * l[k]
            ed = ed + e[k] * d[k]
            dsum = dsum + d[k]
        elig_v[pl.ds(off, 16)] = lsel - r
        return ent_acc + (ed / s - ls) + _EPS * (dsum - 6.0 * ls)

    ent_acc = lax.fori_loop(0, _NG, group, jnp.zeros((16,), jnp.float32))
    ent_v[...] = -ent_acc

    pltpu.sync_copy(elig_v, elig_hbm.at[pl.ds(base, _RPW)])
    pltpu.sync_copy(at_v, at_hbm.at[:, pl.ds(base, _RPW)])
    pltpu.sync_copy(ent_v, ent_hbm.at[pl.ds(wid * 16, 16)])


def _ent_body(p_ref, o_ref):
    o_ref[...] = jnp.sum(p_ref[...]).reshape(1, 1)


def kernel(x, W, b):
    with jax.ensure_compile_time_eval():
        g = jax.random.gumbel(jax.random.key(42), (_N_ROWS, _N_CAT),
                              jnp.float32)
        gt = g.T.copy()

    d0 = pl.pallas_call(
        _mm_body,
        grid=(_N_ROWS // _BT,),
        in_specs=[
            pl.BlockSpec((_N_CAT, _N_FEAT), lambda i: (0, 0)),
            pl.BlockSpec((_N_FEAT, _BT), lambda i: (0, i)),
            pl.BlockSpec((_N_CAT,), lambda i: (0,)),
        ],
        out_specs=pl.BlockSpec((_N_CAT, _BT), lambda i: (0, i)),
        out_shape=jax.ShapeDtypeStruct((_N_CAT, _N_ROWS), jnp.float32),
    )(W, x.T, b)

    mesh = plsc.VectorSubcoreMesh(core_axis_name="c", subcore_axis_name="s")
    elig, at, ent_parts = pl.kernel(
        _sc_body,
        out_type=[
            jax.ShapeDtypeStruct((_N_ROWS,), jnp.float32),
            jax.ShapeDtypeStruct((_N_CAT, _N_ROWS), jnp.float32),
            jax.ShapeDtypeStruct((_NW * 16,), jnp.float32),
        ],
        mesh=mesh,
        compiler_params=pltpu.CompilerParams(needs_layout_passes=False),
        scratch_types=[
            pltpu.VMEM((_N_CAT, _RPW), jnp.float32),
            pltpu.VMEM((_N_CAT, _RPW), jnp.float32),
            pltpu.VMEM((_RPW,), jnp.float32),
            pltpu.VMEM((_N_CAT, _RPW), jnp.float32),
            pltpu.VMEM((16,), jnp.float32),
        ],
    )(d0, gt)

    ent = pl.pallas_call(
        _ent_body,
        in_specs=[pl.BlockSpec((_NW * 16,), lambda: (0,))],
        out_specs=pl.BlockSpec((1, 1), lambda: (0, 0)),
        out_shape=jax.ShapeDtypeStruct((1, 1), jnp.float32),
    )(ent_parts)

    return elig, at.T, ent[0, 0]


# speedup vs baseline: 1.6965x; 1.0014x over previous
import functools

import jax
import jax.numpy as jnp
from jax import lax
from jax.experimental import pallas as pl
from jax.experimental.pallas import tpu as pltpu
from jax.experimental.pallas import tpu_sc as plsc

_EPS = 1e-08
_N_ROWS = 16384
_N_FEAT = 100
_N_CAT = 6
_BT = 2048
_NW = 32
_RPW = _N_ROWS // _NW
_NG = _RPW // 16

_LN2 = 0.6931471805599453


def _mm_body(w_ref, xt_ref, b_ref, d_ref):
    d_ref[...] = lax.dot_general(w_ref[...], xt_ref[...],
                                 (((1,), (0,)), ((), ()))) + b_ref[...][:, None]


def _log_poly(s):
    bits = lax.bitcast_convert_type(s, jnp.int32)
    e = ((bits >> 23) - 127).astype(jnp.float32)
    m = lax.bitcast_convert_type((bits & 0x7FFFFF) | 0x3F800000, jnp.float32)
    u = (m - 1.0) / (m + 1.0)
    u2 = u * u
    poly = 2.0 * u * (1.0 + u2 * (1.0 / 3.0 + u2 * (1.0 / 5.0
                      + u2 * (1.0 / 7.0 + u2 * (1.0 / 9.0)))))
    return poly + e * _LN2


def _sc_body(d_hbm, gt_hbm, elig_hbm, at_hbm, ent_hbm,
             d_v, g_v, elig_v, at_v, ent_v):
    wid = lax.axis_index("s") * 2 + lax.axis_index("c")
    base = wid * _RPW
    pltpu.sync_copy(d_hbm.at[:, pl.ds(base, _RPW)], d_v)
    pltpu.sync_copy(gt_hbm.at[:, pl.ds(base, _RPW)], g_v)

    def group(i, ent_acc):
        off = i * 16
        l = [d_v[k, pl.ds(off, 16)] for k in range(_N_CAT)]
        z = [l[k] + g_v[k, pl.ds(off, 16)] for k in range(_N_CAT)]

        m = l[0]
        for k in range(1, _N_CAT):
            m = jnp.maximum(m, l[k])
        d = [l[k] - m for k in range(_N_CAT)]
        e = [jnp.exp(d[k]) for k in range(_N_CAT)]
        s = e[0]
        for k in range(1, _N_CAT):
            s = s + e[k]
        ls = _log_poly(s)
        r = m + ls

        best = z[0]
        idx = jnp.zeros((16,), jnp.int32)
        for k in range(1, _N_CAT):
            take = z[k] > best
            best = jnp.maximum(best, z[k])
            idx = jnp.where(take, jnp.full((16,), k, jnp.int32), idx)

        lsel = jnp.zeros((16,), jnp.float32)
        ed = jnp.zeros((16,), jnp.float32)
        dsum = jnp.zeros((16,), jnp.float32)
        for k in range(_N_CAT):
            a_k = jnp.where(idx == k, 1.0, 0.0)
            at_v[k, pl.ds(off, 16)] = a_k
            lsel = lsel + a_k * l[k]
            ed = ed + e[k] * d[k]
            dsum = dsum + d[k]
        elig_v[pl.ds(off, 16)] = lsel - r
        return ent_acc + (ed / s - ls) + _EPS * (dsum - 6.0 * ls)

    ent_acc = lax.fori_loop(0, _NG, group, jnp.zeros((16,), jnp.float32))
    ent_v[...] = -ent_acc

    pltpu.sync_copy(elig_v, elig_hbm.at[pl.ds(base, _RPW)])
    pltpu.sync_copy(at_v, at_hbm.at[:, pl.ds(base, _RPW)])
    pltpu.sync_copy(ent_v, ent_hbm.at[pl.ds(wid * 16, 16)])


def _ent_body(p_ref, o_ref):
    o_ref[...] = jnp.sum(p_ref[...]).reshape(1, 1)


def kernel(x, W, b):
    with jax.ensure_compile_time_eval():
        g = jax.random.gumbel(jax.random.key(42), (_N_ROWS, _N_CAT),
                              jnp.float32)
        gt = g.T.copy()

    d0 = pl.pallas_call(
        _mm_body,
        grid=(_N_ROWS // _BT,),
        in_specs=[
            pl.BlockSpec((_N_CAT, _N_FEAT), lambda i: (0, 0)),
            pl.BlockSpec((_N_FEAT, _BT), lambda i: (0, i)),
            pl.BlockSpec((_N_CAT,), lambda i: (0,)),
        ],
        out_specs=pl.BlockSpec((_N_CAT, _BT), lambda i: (0, i)),
        out_shape=jax.ShapeDtypeStruct((_N_CAT, _N_ROWS), jnp.float32),
    )(W, x.T, b)

    mesh = plsc.VectorSubcoreMesh(core_axis_name="c", subcore_axis_name="s")
    elig, at, ent_parts = pl.kernel(
        _sc_body,
        out_type=[
            jax.ShapeDtypeStruct((_N_ROWS,), jnp.float32),
            jax.ShapeDtypeStruct((_N_CAT, _N_ROWS), jnp.float32),
            jax.ShapeDtypeStruct((_NW * 16,), jnp.float32),
        ],
        mesh=mesh,
        compiler_params=pltpu.CompilerParams(needs_layout_passes=False,
                                             skip_device_barrier=True),
        scratch_types=[
            pltpu.VMEM((_N_CAT, _RPW), jnp.float32),
            pltpu.VMEM((_N_CAT, _RPW), jnp.float32),
            pltpu.VMEM((_RPW,), jnp.float32),
            pltpu.VMEM((_N_CAT, _RPW), jnp.float32),
            pltpu.VMEM((16,), jnp.float32),
        ],
    )(d0, gt)

    ent = pl.pallas_call(
        _ent_body,
        in_specs=[pl.BlockSpec((_NW * 16,), lambda: (0,))],
        out_specs=pl.BlockSpec((1, 1), lambda: (0, 0)),
        out_shape=jax.ShapeDtypeStruct((1, 1), jnp.float32),
    )(ent_parts)

    return elig, at.T, ent[0, 0]


# speedup vs baseline: 1.8167x; 1.0709x over previous
import functools

import jax
import jax.numpy as jnp
from jax import lax
from jax.experimental import pallas as pl
from jax.experimental.pallas import tpu as pltpu
from jax.experimental.pallas import tpu_sc as plsc

_EPS = 1e-08
_N_ROWS = 16384
_N_FEAT = 100
_N_CAT = 6
_BT = 2048
_NW = 32
_RPW = _N_ROWS // _NW
_NG = _RPW // 16
_HALF = _N_ROWS // 2

_LN2 = 0.6931471805599453


def _mm_body(w_ref, xt_a_ref, xt_b_ref, b_ref, d_ref):
    w = w_ref[...]
    zero = jnp.zeros_like(w)
    w2 = jnp.concatenate(
        [jnp.concatenate([w, zero], axis=1),
         jnp.concatenate([zero, w], axis=1)], axis=0)
    xt2 = jnp.concatenate([xt_a_ref[...], xt_b_ref[...]], axis=0)
    d2 = lax.dot_general(w2, xt2, (((1,), (0,)), ((), ())))
    bb = b_ref[...][:, None]
    zrow = jnp.zeros((8 - _N_CAT, d2.shape[1]), jnp.float32)
    d_ref[...] = jnp.concatenate(
        [d2[:_N_CAT] + bb, zrow, d2[_N_CAT:] + bb, zrow], axis=0)


def _log_poly(s):
    bits = lax.bitcast_convert_type(s, jnp.int32)
    e = ((bits >> 23) - 127).astype(jnp.float32)
    m = lax.bitcast_convert_type((bits & 0x7FFFFF) | 0x3F800000, jnp.float32)
    u = (m - 1.0) / (m + 1.0)
    u2 = u * u
    poly = 2.0 * u * (1.0 + u2 * (1.0 / 3.0 + u2 * (1.0 / 5.0
                      + u2 * (1.0 / 7.0 + u2 * (1.0 / 9.0)))))
    return poly + e * _LN2


def _sc_body(d_hbm, gt_hbm, elig_hbm, at_hbm, ent_hbm,
             d_v, g_v, elig_v, at_v, ent_v):
    wid = lax.axis_index("s") * 2 + lax.axis_index("c")
    base = wid * _RPW
    krow = (wid >> 4) * 8
    cb = (wid & 15) * _RPW
    pltpu.sync_copy(d_hbm.at[pl.ds(krow, 8), pl.ds(cb, _RPW)], d_v)
    pltpu.sync_copy(gt_hbm.at[pl.ds(krow, 8), pl.ds(cb, _RPW)], g_v)

    def group(i, ent_acc):
        off = i * 16
        l = [d_v[k, pl.ds(off, 16)] for k in range(_N_CAT)]
        z = [l[k] + g_v[k, pl.ds(off, 16)] for k in range(_N_CAT)]

        m = l[0]
        for k in range(1, _N_CAT):
            m = jnp.maximum(m, l[k])
        d = [l[k] - m for k in range(_N_CAT)]
        e = [jnp.exp(d[k]) for k in range(_N_CAT)]
        s = e[0]
        for k in range(1, _N_CAT):
            s = s + e[k]
        ls = _log_poly(s)
        r = m + ls

        best = z[0]
        idx = jnp.zeros((16,), jnp.int32)
        for k in range(1, _N_CAT):
            take = z[k] > best
            best = jnp.maximum(best, z[k])
            idx = jnp.where(take, jnp.full((16,), k, jnp.int32), idx)

        lsel = jnp.zeros((16,), jnp.float32)
        ed = jnp.zeros((16,), jnp.float32)
        dsum = jnp.zeros((16,), jnp.float32)
        for k in range(_N_CAT):
            a_k = jnp.where(idx == k, 1.0, 0.0)
            at_v[k, pl.ds(off, 16)] = a_k
            lsel = lsel + a_k * l[k]
            ed = ed + e[k] * d[k]
            dsum = dsum + d[k]
        elig_v[pl.ds(off, 16)] = lsel - r
        return ent_acc + (ed / s - ls) + _EPS * (dsum - 6.0 * ls)

    ent_acc = lax.fori_loop(0, _NG, group, jnp.zeros((16,), jnp.float32))
    ent_v[...] = -ent_acc

    pltpu.sync_copy(elig_v, elig_hbm.at[pl.ds(base, _RPW)])
    pltpu.sync_copy(at_v, at_hbm.at[:, pl.ds(base, _RPW)])
    pltpu.sync_copy(ent_v, ent_hbm.at[pl.ds(wid * 16, 16)])


def _ent_body(p_ref, o_ref):
    o_ref[...] = jnp.sum(p_ref[...]).reshape(1, 1)


def kernel(x, W, b):
    with jax.ensure_compile_time_eval():
        g = jax.random.gumbel(jax.random.key(42), (_N_ROWS, _N_CAT),
                              jnp.float32)
        gt = g.T.copy()
        zp = jnp.zeros((8 - _N_CAT, _HALF), jnp.float32)
        gt2 = jnp.concatenate(
            [gt[:, :_HALF], zp, gt[:, _HALF:], zp], axis=0).copy()

    xt = x.T
    d0 = pl.pallas_call(
        _mm_body,
        grid=(_HALF // _BT,),
        in_specs=[
            pl.BlockSpec((_N_CAT, _N_FEAT), lambda i: (0, 0)),
            pl.BlockSpec((_N_FEAT, _BT), lambda i: (0, i)),
            pl.BlockSpec((_N_FEAT, _BT), lambda i: (0, i + _HALF // _BT)),
            pl.BlockSpec((_N_CAT,), lambda i: (0,)),
        ],
        out_specs=pl.BlockSpec((16, _BT), lambda i: (0, i)),
        out_shape=jax.ShapeDtypeStruct((16, _HALF), jnp.float32),
    )(W, xt, xt, b)

    mesh = plsc.VectorSubcoreMesh(core_axis_name="c", subcore_axis_name="s")
    elig, at, ent_parts = pl.kernel(
        _sc_body,
        out_type=[
            jax.ShapeDtypeStruct((_N_ROWS,), jnp.float32),
            jax.ShapeDtypeStruct((_N_CAT, _N_ROWS), jnp.float32),
            jax.ShapeDtypeStruct((_NW * 16,), jnp.float32),
        ],
        mesh=mesh,
        compiler_params=pltpu.CompilerParams(needs_layout_passes=False,
                                             skip_device_barrier=True),
        scratch_types=[
            pltpu.VMEM((8, _RPW), jnp.float32),
            pltpu.VMEM((8, _RPW), jnp.float32),
            pltpu.VMEM((_RPW,), jnp.float32),
            pltpu.VMEM((_N_CAT, _RPW), jnp.float32),
            pltpu.VMEM((16,), jnp.float32),
        ],
    )(d0, gt2)

    ent = pl.pallas_call(
        _ent_body,
        in_specs=[pl.BlockSpec((_NW * 16,), lambda: (0,))],
        out_specs=pl.BlockSpec((1, 1), lambda: (0, 0)),
        out_shape=jax.ShapeDtypeStruct((1, 1), jnp.float32),
    )(ent_parts)

    return elig, at.T, ent[0, 0]


# speedup vs baseline: 1.8781x; 1.0338x over previous
import functools

import jax
import jax.numpy as jnp
from jax import lax
from jax.experimental import pallas as pl
from jax.experimental.pallas import tpu as pltpu
from jax.experimental.pallas import tpu_sc as plsc

_EPS = 1e-08
_N_ROWS = 16384
_N_FEAT = 100
_N_CAT = 6
_BT = 4096
_NW = 32
_RPW = _N_ROWS // _NW
_NG = _RPW // 16
_HALF = _N_ROWS // 2

_LN2 = 0.6931471805599453


def _mm_body(w_ref, xt_a_ref, xt_b_ref, b_ref, d_ref):
    w = w_ref[...]
    zero = jnp.zeros_like(w)
    w2 = jnp.concatenate(
        [jnp.concatenate([w, zero], axis=1),
         jnp.concatenate([zero, w], axis=1)], axis=0)
    xt2 = jnp.concatenate([xt_a_ref[...], xt_b_ref[...]], axis=0)
    d2 = lax.dot_general(w2, xt2, (((1,), (0,)), ((), ())))
    bb = b_ref[...][:, None]
    zrow = jnp.zeros((8 - _N_CAT, d2.shape[1]), jnp.float32)
    d_ref[...] = jnp.concatenate(
        [d2[:_N_CAT] + bb, zrow, d2[_N_CAT:] + bb, zrow], axis=0)


def _log_poly(s):
    bits = lax.bitcast_convert_type(s, jnp.int32)
    e = ((bits >> 23) - 127).astype(jnp.float32)
    m = lax.bitcast_convert_type((bits & 0x7FFFFF) | 0x3F800000, jnp.float32)
    u = (m - 1.0) / (m + 1.0)
    u2 = u * u
    poly = 2.0 * u * (1.0 + u2 * (1.0 / 3.0 + u2 * (1.0 / 5.0
                      + u2 * (1.0 / 7.0 + u2 * (1.0 / 9.0)))))
    return poly + e * _LN2


def _sc_body(d_hbm, gt_hbm, elig_hbm, at_hbm, ent_hbm,
             d_v, g_v, elig_v, at_v, ent_v):
    wid = lax.axis_index("s") * 2 + lax.axis_index("c")
    base = wid * _RPW
    krow = (wid >> 4) * 8
    cb = (wid & 15) * _RPW
    pltpu.sync_copy(d_hbm.at[pl.ds(krow, 8), pl.ds(cb, _RPW)], d_v)
    pltpu.sync_copy(gt_hbm.at[pl.ds(krow, 8), pl.ds(cb, _RPW)], g_v)

    def group(i, ent_acc):
        off = i * 16
        l = [d_v[k, pl.ds(off, 16)] for k in range(_N_CAT)]
        z = [l[k] + g_v[k, pl.ds(off, 16)] for k in range(_N_CAT)]

        m = l[0]
        for k in range(1, _N_CAT):
            m = jnp.maximum(m, l[k])
        d = [l[k] - m for k in range(_N_CAT)]
        e = [jnp.exp(d[k]) for k in range(_N_CAT)]
        s = e[0]
        for k in range(1, _N_CAT):
            s = s + e[k]
        ls = _log_poly(s)
        r = m + ls

        best = z[0]
        idx = jnp.zeros((16,), jnp.int32)
        for k in range(1, _N_CAT):
            take = z[k] > best
            best = jnp.maximum(best, z[k])
            idx = jnp.where(take, jnp.full((16,), k, jnp.int32), idx)

        lsel = jnp.zeros((16,), jnp.float32)
        ed = jnp.zeros((16,), jnp.float32)
        dsum = jnp.zeros((16,), jnp.float32)
        for k in range(_N_CAT):
            a_k = jnp.where(idx == k, 1.0, 0.0)
            at_v[k, pl.ds(off, 16)] = a_k
            lsel = lsel + a_k * l[k]
            ed = ed + e[k] * d[k]
            dsum = dsum + d[k]
        elig_v[pl.ds(off, 16)] = lsel - r
        return ent_acc + (ed / s - ls) + _EPS * (dsum - 6.0 * ls)

    ent_acc = lax.fori_loop(0, _NG, group, jnp.zeros((16,), jnp.float32))
    ent_v[...] = -ent_acc

    pltpu.sync_copy(elig_v, elig_hbm.at[pl.ds(base, _RPW)])
    pltpu.sync_copy(at_v, at_hbm.at[:, pl.ds(base, _RPW)])
    pltpu.sync_copy(ent_v, ent_hbm.at[pl.ds(wid * 16, 16)])


def _ent_body(p_ref, o_ref):
    o_ref[...] = jnp.sum(p_ref[...]).reshape(1, 1)


def kernel(x, W, b):
    with jax.ensure_compile_time_eval():
        g = jax.random.gumbel(jax.random.key(42), (_N_ROWS, _N_CAT),
                              jnp.float32)
        gt = g.T.copy()
        zp = jnp.zeros((8 - _N_CAT, _HALF), jnp.float32)
        gt2 = jnp.concatenate(
            [gt[:, :_HALF], zp, gt[:, _HALF:], zp], axis=0).copy()

    xt = x.T
    d0 = pl.pallas_call(
        _mm_body,
        grid=(_HALF // _BT,),
        in_specs=[
            pl.BlockSpec((_N_CAT, _N_FEAT), lambda i: (0, 0)),
            pl.BlockSpec((_N_FEAT, _BT), lambda i: (0, i)),
            pl.BlockSpec((_N_FEAT, _BT), lambda i: (0, i + _HALF // _BT)),
            pl.BlockSpec((_N_CAT,), lambda i: (0,)),
        ],
        out_specs=pl.BlockSpec((16, _BT), lambda i: (0, i)),
        out_shape=jax.ShapeDtypeStruct((16, _HALF), jnp.float32),
    )(W, xt, xt, b)

    mesh = plsc.VectorSubcoreMesh(core_axis_name="c", subcore_axis_name="s")
    elig, at, ent_parts = pl.kernel(
        _sc_body,
        out_type=[
            jax.ShapeDtypeStruct((_N_ROWS,), jnp.float32),
            jax.ShapeDtypeStruct((_N_CAT, _N_ROWS), jnp.float32),
            jax.ShapeDtypeStruct((_NW * 16,), jnp.float32),
        ],
        mesh=mesh,
        compiler_params=pltpu.CompilerParams(needs_layout_passes=False,
                                             skip_device_barrier=True),
        scratch_types=[
            pltpu.VMEM((8, _RPW), jnp.float32),
            pltpu.VMEM((8, _RPW), jnp.float32),
            pltpu.VMEM((_RPW,), jnp.float32),
            pltpu.VMEM((_N_CAT, _RPW), jnp.float32),
            pltpu.VMEM((16,), jnp.float32),
        ],
    )(d0, gt2)

    ent = pl.pallas_call(
        _ent_body,
        in_specs=[pl.BlockSpec((_NW * 16,), lambda: (0,))],
        out_specs=pl.BlockSpec((1, 1), lambda: (0, 0)),
        out_shape=jax.ShapeDtypeStruct((1, 1), jnp.float32),
    )(ent_parts)

    return elig, at.T, ent[0, 0]
